# baseline (speedup 1.0000x reference)
import functools

import jax
import jax.numpy as jnp
from jax import lax
from jax.experimental import pallas as pl
from jax.experimental.pallas import tpu as pltpu
from jax.experimental.pallas import tpu_sc as plsc

N_NODES = 10000
NFEAT = 128
NHID = 64
NCLASS = 16
K_STEPS = 10
ALPHA = 0.1

NS = 16
CB = 512
ROWS_PER_TILE = 640
PAD_ROWS = NS * ROWS_PER_TILE
ZR = 64
RSQRT_MAGIC = 0x5F3759DF


def _mlp_body(x_ref, w1_ref, b1_ref, w2_ref, b2_ref, o_ref):
    h = jnp.dot(x_ref[...], w1_ref[...], preferred_element_type=jnp.float32)
    h = jnp.maximum(h + b1_ref[...], 0.0)
    o_ref[...] = (
        jnp.dot(h, w2_ref[...], preferred_element_type=jnp.float32) + b2_ref[...]
    )


def _mlp(x, w1t, b1, w2t, b2):
    return pl.pallas_call(
        _mlp_body,
        out_shape=jax.ShapeDtypeStruct((N_NODES, NCLASS), jnp.float32),
    )(x, w1t, b1.reshape(1, NHID), w2t, b2.reshape(1, NCLASS))


def _appnp_sc(z, src_pad, dst_pad, nchunk):
    mesh = plsc.VectorSubcoreMesh(
        core_axis_name="c", subcore_axis_name="s", num_cores=2, num_subcores=NS
    )

    @functools.partial(
        pl.kernel,
        out_type=jax.ShapeDtypeStruct((PAD_ROWS, NCLASS), jnp.float32),
        mesh=mesh,
        compiler_params=pltpu.CompilerParams(
            needs_layout_passes=False, use_tc_tiling_on_sc=False
        ),
        scratch_types=[
            pltpu.VMEM_SHARED((PAD_ROWS, NCLASS), jnp.float32),
            pltpu.VMEM_SHARED((PAD_ROWS, NCLASS), jnp.float32),
            pltpu.VMEM((nchunk, CB), jnp.int32),
            pltpu.VMEM((nchunk, CB), jnp.int32),
            pltpu.VMEM((ROWS_PER_TILE, NCLASS), jnp.float32),
            pltpu.VMEM((ROWS_PER_TILE, NCLASS), jnp.float32),
            pltpu.VMEM((ZR, NCLASS), jnp.float32),
            pltpu.VMEM((ROWS_PER_TILE, NCLASS), jnp.float32),
            pltpu.VMEM((ROWS_PER_TILE, NCLASS), jnp.float32),
            pltpu.VMEM((ROWS_PER_TILE, NCLASS), jnp.float32),
            pltpu.VMEM((ROWS_PER_TILE, NCLASS), jnp.float32),
            pltpu.SemaphoreType.DMA,
            pltpu.SemaphoreType.DMA,
            pltpu.SemaphoreType.DMA,
            pltpu.SemaphoreType.DMA,
        ],
    )
    def k(z_hbm, srcp_hbm, dstp_hbm, out_hbm, outa_sh, s_sh, src_t, dst_t,
          g_a, g_b, zer_t, a_t, c1_t, c2_t, dis_t,
          sem_ga, sem_gb, sem_sa, sem_sb):
        cid = lax.axis_index("c")
        sid = lax.axis_index("s")

        @pl.when(cid == 0)
        def _():
            rbase = sid * ROWS_PER_TILE
            rows = pl.ds(rbase, ROWS_PER_TILE)
            gsl_a = g_a.at[pl.ds(0, CB)]
            gsl_b = g_b.at[pl.ds(0, CB)]

            pltpu.sync_copy(srcp_hbm.at[sid], src_t)
            pltpu.sync_copy(dstp_hbm.at[sid], dst_t)

            @pl.loop(0, ZR)
            def _(i):
                zer_t[i, :] = jnp.full((NCLASS,), 0.0, jnp.float32)

            @pl.loop(0, CB)
            def _(i):
                g_a[i, :] = jnp.full((NCLASS,), 1.0, jnp.float32)

            @pl.loop(0, ROWS_PER_TILE // ZR)
            def _(t):
                pltpu.sync_copy(zer_t, s_sh.at[pl.ds(rbase + t * ZR, ZR)])

            plsc.subcore_barrier()

            @pl.loop(0, nchunk, step=8)
            def _(j):
                for b in range(8):
                    pltpu.async_copy(
                        gsl_a, s_sh.at[dst_t.at[j + b]], sem_sa, add=True
                    )
                for b in range(8):
                    pltpu.make_async_copy(
                        gsl_a, s_sh.at[dst_t.at[j + b]], sem_sa
                    ).wait()

            plsc.subcore_barrier()

            pltpu.sync_copy(s_sh.at[rows], g_a)
            pltpu.sync_copy(z_hbm.at[rows], g_b)

            @pl.loop(0, ROWS_PER_TILE)
            def _(i):
                d = g_a[i, :] + 1.0
                ihalf = plsc.bitcast(d, jnp.int32) >> 1
                y = plsc.bitcast(
                    jnp.full((NCLASS,), RSQRT_MAGIC, jnp.int32) - ihalf,
                    jnp.float32,
                )
                y = y * (1.5 - 0.5 * d * y * y)
                y = y * (1.5 - 0.5 * d * y * y)
                y = y * (1.5 - 0.5 * d * y * y)
                zrow = g_b[i, :]
                dis_t[i, :] = y
                c1_t[i, :] = (1.0 - ALPHA) * y * y
                c2_t[i, :] = ALPHA * y * zrow
                a_t[i, :] = y * zrow

            pltpu.sync_copy(a_t, outa_sh.at[rows])

            @pl.loop(0, ROWS_PER_TILE // ZR)
            def _(t):
                pltpu.sync_copy(zer_t, s_sh.at[pl.ds(rbase + t * ZR, ZR)])

            plsc.subcore_barrier()

            @pl.loop(0, K_STEPS)
            def _(_k):
                pltpu.async_copy(outa_sh.at[src_t.at[0]], gsl_a, sem_ga)

                @pl.loop(0, nchunk, step=2)
                def _(j):
                    pltpu.make_async_copy(
                        outa_sh.at[src_t.at[j]], gsl_a, sem_ga
                    ).wait()
                    pltpu.async_copy(
                        gsl_a, s_sh.at[dst_t.at[j]], sem_sa, add=True
                    )
                    pltpu.async_copy(outa_sh.at[src_t.at[j + 1]], gsl_b, sem_gb)
                    pltpu.make_async_copy(
                        outa_sh.at[src_t.at[j + 1]], gsl_b, sem_gb
                    ).wait()
                    pltpu.make_async_copy(
                        gsl_a, s_sh.at[dst_t.at[j]], sem_sa
                    ).wait()
                    pltpu.async_copy(
                        gsl_b, s_sh.at[dst_t.at[j + 1]], sem_sb, add=True
                    )
                    jn = jnp.minimum(j + 2, nchunk - 1)
                    pltpu.async_copy(outa_sh.at[src_t.at[jn]], gsl_a, sem_ga)
                    pltpu.make_async_copy(
                        gsl_b, s_sh.at[dst_t.at[j + 1]], sem_sb
                    ).wait()

                pltpu.make_async_copy(
                    outa_sh.at[src_t.at[0]], gsl_a, sem_ga
                ).wait()
                plsc.subcore_barrier()

                pltpu.sync_copy(s_sh.at[rows], g_a)

                @pl.loop(0, ROWS_PER_TILE)
                def _(i):
                    a_t[i, :] = c1_t[i, :] * (g_a[i, :] + a_t[i, :]) + c2_t[i, :]

                pltpu.sync_copy(a_t, outa_sh.at[rows])

                @pl.loop(0, ROWS_PER_TILE // ZR)
                def _(t):
                    pltpu.sync_copy(zer_t, s_sh.at[pl.ds(rbase + t * ZR, ZR)])

                plsc.subcore_barrier()

            @pl.loop(0, ROWS_PER_TILE)
            def _(i):
                a_t[i, :] = a_t[i, :] / dis_t[i, :]

            pltpu.sync_copy(a_t, out_hbm.at[rows])

    return k(z, src_pad, dst_pad)


def kernel(x, edge_index, W1, b1, W2, b2):
    z = _mlp(x, W1.T, b1, W2.T, b2)
    z = jnp.pad(z, ((0, PAD_ROWS - N_NODES), (0, 0)))

    e = edge_index.shape[1]
    nchunk = -(-e // (NS * CB))
    nchunk += nchunk % 2
    ep = nchunk * NS * CB
    npad = ep - e
    pad_idx = N_NODES + (jnp.arange(npad, dtype=jnp.int32) % (PAD_ROWS - N_NODES))
    src_pad = jnp.concatenate([edge_index[0], pad_idx]).reshape(NS, nchunk, CB)
    dst_pad = jnp.concatenate([edge_index[1], pad_idx]).reshape(NS, nchunk, CB)

    return _appnp_sc(z, src_pad, dst_pad, nchunk)[:N_NODES]


# speedup vs baseline: 1.0653x; 1.0653x over previous
import functools

import jax
import jax.numpy as jnp
from jax import lax
from jax.experimental import pallas as pl
from jax.experimental.pallas import tpu as pltpu
from jax.experimental.pallas import tpu_sc as plsc

N_NODES = 10000
NFEAT = 128
NHID = 64
NCLASS = 16
K_STEPS = 10
ALPHA = 0.1

NS = 16
CB = 512
ROWS_PER_TILE = 640
PAD_ROWS = NS * ROWS_PER_TILE
ZR = 64
RSQRT_MAGIC = 0x5F3759DF


def _mlp_body(x_ref, w1_ref, b1_ref, w2_ref, b2_ref, o_ref):
    h = jnp.dot(x_ref[...], w1_ref[...], preferred_element_type=jnp.float32)
    h = jnp.maximum(h + b1_ref[...], 0.0)
    o_ref[...] = (
        jnp.dot(h, w2_ref[...], preferred_element_type=jnp.float32) + b2_ref[...]
    )


def _mlp(x, w1t, b1, w2t, b2):
    return pl.pallas_call(
        _mlp_body,
        out_shape=jax.ShapeDtypeStruct((N_NODES, NCLASS), jnp.float32),
    )(x, w1t, b1.reshape(1, NHID), w2t, b2.reshape(1, NCLASS))


def _appnp_sc(z, src_pad, dst_pad, nchunk):
    mesh = plsc.VectorSubcoreMesh(
        core_axis_name="c", subcore_axis_name="s", num_cores=2, num_subcores=NS
    )

    @functools.partial(
        pl.kernel,
        out_type=jax.ShapeDtypeStruct((PAD_ROWS, NCLASS), jnp.float32),
        mesh=mesh,
        compiler_params=pltpu.CompilerParams(
            needs_layout_passes=False, use_tc_tiling_on_sc=False
        ),
        scratch_types=[
            pltpu.VMEM_SHARED((PAD_ROWS, NCLASS), jnp.float32),
            pltpu.VMEM_SHARED((PAD_ROWS, NCLASS), jnp.float32),
            pltpu.VMEM((nchunk, CB), jnp.int32),
            pltpu.VMEM((nchunk, CB), jnp.int32),
            pltpu.VMEM((ROWS_PER_TILE, NCLASS), jnp.float32),
            pltpu.VMEM((ROWS_PER_TILE, NCLASS), jnp.float32),
            pltpu.VMEM((ZR, NCLASS), jnp.float32),
            pltpu.VMEM((ROWS_PER_TILE, NCLASS), jnp.float32),
            pltpu.VMEM((ROWS_PER_TILE, NCLASS), jnp.float32),
            pltpu.VMEM((ROWS_PER_TILE, NCLASS), jnp.float32),
            pltpu.VMEM((ROWS_PER_TILE, NCLASS), jnp.float32),
            pltpu.SemaphoreType.DMA,
            pltpu.SemaphoreType.DMA,
            pltpu.SemaphoreType.DMA,
            pltpu.SemaphoreType.DMA,
        ],
    )
    def k(z_hbm, srcp_hbm, dstp_hbm, out_hbm, outa_sh, s_sh, src_t, dst_t,
          g_a, g_b, zer_t, a_t, c1_t, c2_t, dis_t,
          sem_ga, sem_gb, sem_sa, sem_sb):
        cid = lax.axis_index("c")
        sid = lax.axis_index("s")

        @pl.when(cid == 0)
        def _():
            rbase = sid * ROWS_PER_TILE
            rows = pl.ds(rbase, ROWS_PER_TILE)
            gsl_a = g_a.at[pl.ds(0, CB)]
            gsl_b = g_b.at[pl.ds(0, CB)]

            pltpu.sync_copy(srcp_hbm.at[sid], src_t)
            pltpu.sync_copy(dstp_hbm.at[sid], dst_t)

            @pl.loop(0, ZR)
            def _(i):
                zer_t[i, :] = jnp.full((NCLASS,), 0.0, jnp.float32)

            @pl.loop(0, CB)
            def _(i):
                g_a[i, :] = jnp.full((NCLASS,), 1.0, jnp.float32)

            @pl.loop(0, ROWS_PER_TILE // ZR)
            def _(t):
                pltpu.sync_copy(zer_t, s_sh.at[pl.ds(rbase + t * ZR, ZR)])

            plsc.subcore_barrier()

            @pl.loop(0, nchunk, step=8)
            def _(j):
                for b in range(8):
                    pltpu.async_copy(
                        gsl_a, s_sh.at[dst_t.at[j + b]], sem_sa, add=True
                    )
                for b in range(8):
                    pltpu.make_async_copy(
                        gsl_a, s_sh.at[dst_t.at[j + b]], sem_sa
                    ).wait()

            plsc.subcore_barrier()

            pltpu.sync_copy(s_sh.at[rows], g_a)
            pltpu.sync_copy(z_hbm.at[rows], g_b)

            @pl.loop(0, ROWS_PER_TILE, step=2)
            def _(i0):
              for i in (i0, i0 + 1):
                d = g_a[i, :] + 1.0
                ihalf = plsc.bitcast(d, jnp.int32) >> 1
                y = plsc.bitcast(
                    jnp.full((NCLASS,), RSQRT_MAGIC, jnp.int32) - ihalf,
                    jnp.float32,
                )
                y = y * (1.5 - 0.5 * d * y * y)
                y = y * (1.5 - 0.5 * d * y * y)
                y = y * (1.5 - 0.5 * d * y * y)
                zrow = g_b[i, :]
                dis_t[i, :] = y
                c1_t[i, :] = (1.0 - ALPHA) * y * y
                c2_t[i, :] = ALPHA * y * zrow
                a_t[i, :] = y * zrow

            pltpu.sync_copy(a_t, outa_sh.at[rows])

            @pl.loop(0, ROWS_PER_TILE // ZR)
            def _(t):
                pltpu.sync_copy(zer_t, s_sh.at[pl.ds(rbase + t * ZR, ZR)])

            plsc.subcore_barrier()

            @pl.loop(0, K_STEPS)
            def _(_k):
                pltpu.async_copy(outa_sh.at[src_t.at[0]], gsl_a, sem_ga)

                @pl.loop(0, nchunk, step=2)
                def _(j):
                    pltpu.make_async_copy(
                        outa_sh.at[src_t.at[j]], gsl_a, sem_ga
                    ).wait()
                    pltpu.async_copy(
                        gsl_a, s_sh.at[dst_t.at[j]], sem_sa, add=True
                    )
                    pltpu.async_copy(outa_sh.at[src_t.at[j + 1]], gsl_b, sem_gb)
                    pltpu.make_async_copy(
                        outa_sh.at[src_t.at[j + 1]], gsl_b, sem_gb
                    ).wait()
                    pltpu.make_async_copy(
                        gsl_a, s_sh.at[dst_t.at[j]], sem_sa
                    ).wait()
                    pltpu.async_copy(
                        gsl_b, s_sh.at[dst_t.at[j + 1]], sem_sb, add=True
                    )
                    jn = jnp.minimum(j + 2, nchunk - 1)
                    pltpu.async_copy(outa_sh.at[src_t.at[jn]], gsl_a, sem_ga)
                    pltpu.make_async_copy(
                        gsl_b, s_sh.at[dst_t.at[j + 1]], sem_sb
                    ).wait()

                pltpu.make_async_copy(
                    outa_sh.at[src_t.at[0]], gsl_a, sem_ga
                ).wait()
                plsc.subcore_barrier()

                pltpu.sync_copy(s_sh.at[rows], g_a)

                @pl.loop(0, ROWS_PER_TILE, step=8)
                def _(i0):
                    for b in range(8):
                        i = i0 + b
                        a_t[i, :] = (
                            c1_t[i, :] * (g_a[i, :] + a_t[i, :]) + c2_t[i, :]
                        )

                pltpu.sync_copy(a_t, outa_sh.at[rows])

                @pl.loop(0, ROWS_PER_TILE // ZR)
                def _(t):
                    pltpu.sync_copy(zer_t, s_sh.at[pl.ds(rbase + t * ZR, ZR)])

                plsc.subcore_barrier()

            @pl.loop(0, ROWS_PER_TILE)
            def _(i):
                a_t[i, :] = a_t[i, :] / dis_t[i, :]

            pltpu.sync_copy(a_t, out_hbm.at[rows])

    return k(z, src_pad, dst_pad)


def kernel(x, edge_index, W1, b1, W2, b2):
    z = _mlp(x, W1.T, b1, W2.T, b2)
    z = jnp.pad(z, ((0, PAD_ROWS - N_NODES), (0, 0)))

    e = edge_index.shape[1]
    nchunk = -(-e // (NS * CB))
    nchunk += nchunk % 2
    ep = nchunk * NS * CB
    npad = ep - e
    pad_idx = N_NODES + (jnp.arange(npad, dtype=jnp.int32) % (PAD_ROWS - N_NODES))
    src_pad = jnp.concatenate([edge_index[0], pad_idx]).reshape(NS, nchunk, CB)
    dst_pad = jnp.concatenate([edge_index[1], pad_idx]).reshape(NS, nchunk, CB)

    return _appnp_sc(z, src_pad, dst_pad, nchunk)[:N_NODES]


# speedup vs baseline: 1.0945x; 1.0274x over previous
import functools

import jax
import jax.numpy as jnp
from jax import lax
from jax.experimental import pallas as pl
from jax.experimental.pallas import tpu as pltpu
from jax.experimental.pallas import tpu_sc as plsc

N_NODES = 10000
NFEAT = 128
NHID = 64
NCLASS = 16
K_STEPS = 10
ALPHA = 0.1

NS = 16
CB = 512
ROWS_PER_TILE = 640
PAD_ROWS = NS * ROWS_PER_TILE
ZR = 64
RSQRT_MAGIC = 0x5F3759DF


def _mlp_body(x_ref, w1_ref, b1_ref, w2_ref, b2_ref, o_ref):
    h = jnp.dot(x_ref[...], w1_ref[...], preferred_element_type=jnp.float32)
    h = jnp.maximum(h + b1_ref[...], 0.0)
    o_ref[...] = (
        jnp.dot(h, w2_ref[...], preferred_element_type=jnp.float32) + b2_ref[...]
    )


def _mlp(x, w1t, b1, w2t, b2):
    return pl.pallas_call(
        _mlp_body,
        out_shape=jax.ShapeDtypeStruct((N_NODES, NCLASS), jnp.float32),
    )(x, w1t, b1.reshape(1, NHID), w2t, b2.reshape(1, NCLASS))


def _appnp_sc(z, src_pad, dst_pad, nchunk):
    mesh = plsc.VectorSubcoreMesh(
        core_axis_name="c", subcore_axis_name="s", num_cores=2, num_subcores=NS
    )

    @functools.partial(
        pl.kernel,
        out_type=jax.ShapeDtypeStruct((PAD_ROWS, NCLASS), jnp.float32),
        mesh=mesh,
        compiler_params=pltpu.CompilerParams(
            needs_layout_passes=False, use_tc_tiling_on_sc=False
        ),
        scratch_types=[
            pltpu.VMEM_SHARED((PAD_ROWS, NCLASS), jnp.float32),
            pltpu.VMEM_SHARED((PAD_ROWS, NCLASS), jnp.float32),
            pltpu.VMEM((nchunk, CB), jnp.int32),
            pltpu.VMEM((nchunk, CB), jnp.int32),
            pltpu.VMEM((ROWS_PER_TILE, NCLASS), jnp.float32),
            pltpu.VMEM((ROWS_PER_TILE, NCLASS), jnp.float32),
            pltpu.VMEM((ZR, NCLASS), jnp.float32),
            pltpu.VMEM((ROWS_PER_TILE, NCLASS), jnp.float32),
            pltpu.VMEM((ROWS_PER_TILE, NCLASS), jnp.float32),
            pltpu.VMEM((ROWS_PER_TILE, NCLASS), jnp.float32),
            pltpu.VMEM((ROWS_PER_TILE, NCLASS), jnp.float32),
            pltpu.VMEM_SHARED((PAD_ROWS,), jnp.float32),
            pltpu.VMEM((CB,), jnp.float32),
            pltpu.VMEM((ROWS_PER_TILE,), jnp.float32),
            pltpu.SemaphoreType.DMA,
            pltpu.SemaphoreType.DMA,
            pltpu.SemaphoreType.DMA,
            pltpu.SemaphoreType.DMA,
        ],
    )
    def k(z_hbm, srcp_hbm, dstp_hbm, out_hbm, outa_sh, s_sh, src_t, dst_t,
          g_a, g_b, zer_t, a_t, c1_t, c2_t, dis_t, deg_sh, ones1, dstage,
          sem_ga, sem_gb, sem_sa, sem_sb):
        cid = lax.axis_index("c")
        sid = lax.axis_index("s")

        @pl.when(cid == 0)
        def _():
            rbase = sid * ROWS_PER_TILE
            rows = pl.ds(rbase, ROWS_PER_TILE)
            gsl_a = g_a.at[pl.ds(0, CB)]
            gsl_b = g_b.at[pl.ds(0, CB)]

            pltpu.sync_copy(srcp_hbm.at[sid], src_t)
            pltpu.sync_copy(dstp_hbm.at[sid], dst_t)

            @pl.loop(0, ZR)
            def _(i):
                zer_t[i, :] = jnp.full((NCLASS,), 0.0, jnp.float32)

            @pl.loop(0, CB // 16)
            def _(i):
                ones1[pl.ds(i * 16, 16)] = jnp.full((16,), 1.0, jnp.float32)

            @pl.loop(0, ROWS_PER_TILE // 16)
            def _(i):
                dstage[pl.ds(i * 16, 16)] = jnp.full((16,), 0.0, jnp.float32)

            pltpu.sync_copy(dstage, deg_sh.at[rows])

            @pl.loop(0, ROWS_PER_TILE // ZR)
            def _(t):
                pltpu.sync_copy(zer_t, s_sh.at[pl.ds(rbase + t * ZR, ZR)])

            plsc.subcore_barrier()

            @pl.loop(0, nchunk, step=8)
            def _(j):
                for b in range(8):
                    pltpu.async_copy(
                        ones1, deg_sh.at[dst_t.at[j + b]], sem_sa, add=True
                    )
                for b in range(8):
                    pltpu.make_async_copy(
                        ones1, deg_sh.at[dst_t.at[j + b]], sem_sa
                    ).wait()

            plsc.subcore_barrier()

            pltpu.sync_copy(deg_sh.at[rows], dstage)
            pltpu.sync_copy(z_hbm.at[rows], g_b)

            @pl.loop(0, ROWS_PER_TILE // 16)
            def _(g):
                d = dstage[pl.ds(g * 16, 16)] + 1.0
                ihalf = plsc.bitcast(d, jnp.int32) >> 1
                y = plsc.bitcast(
                    jnp.full((16,), RSQRT_MAGIC, jnp.int32) - ihalf,
                    jnp.float32,
                )
                y = y * (1.5 - 0.5 * d * y * y)
                y = y * (1.5 - 0.5 * d * y * y)
                y = y * (1.5 - 0.5 * d * y * y)
                lane = lax.iota(jnp.int32, 16)
                for b in range(16):
                    onehot = jnp.where(lane == b, 1.0, 0.0).astype(jnp.float32)
                    yb = jnp.sum(y * onehot)
                    i = g * 16 + b
                    zrow = g_b[i, :]
                    dis_t[i, :] = jnp.full((NCLASS,), 1.0, jnp.float32) * yb
                    c1_t[i, :] = dis_t[i, :] * ((1.0 - ALPHA) * yb)
                    c2_t[i, :] = (ALPHA * yb) * zrow
                    a_t[i, :] = yb * zrow

            pltpu.sync_copy(a_t, outa_sh.at[rows])
            plsc.subcore_barrier()

            @pl.loop(0, K_STEPS)
            def _(_k):
                pltpu.async_copy(outa_sh.at[src_t.at[0]], gsl_a, sem_ga)

                @pl.loop(0, nchunk, step=2)
                def _(j):
                    pltpu.make_async_copy(
                        outa_sh.at[src_t.at[j]], gsl_a, sem_ga
                    ).wait()
                    pltpu.async_copy(
                        gsl_a, s_sh.at[dst_t.at[j]], sem_sa, add=True
                    )
                    pltpu.async_copy(outa_sh.at[src_t.at[j + 1]], gsl_b, sem_gb)
                    pltpu.make_async_copy(
                        outa_sh.at[src_t.at[j + 1]], gsl_b, sem_gb
                    ).wait()
                    pltpu.make_async_copy(
                        gsl_a, s_sh.at[dst_t.at[j]], sem_sa
                    ).wait()
                    pltpu.async_copy(
                        gsl_b, s_sh.at[dst_t.at[j + 1]], sem_sb, add=True
                    )
                    jn = jnp.minimum(j + 2, nchunk - 1)
                    pltpu.async_copy(outa_sh.at[src_t.at[jn]], gsl_a, sem_ga)
                    pltpu.make_async_copy(
                        gsl_b, s_sh.at[dst_t.at[j + 1]], sem_sb
                    ).wait()

                pltpu.make_async_copy(
                    outa_sh.at[src_t.at[0]], gsl_a, sem_ga
                ).wait()
                plsc.subcore_barrier()

                pltpu.sync_copy(s_sh.at[rows], g_a)

                @pl.loop(0, ROWS_PER_TILE, step=8)
                def _(i0):
                    for b in range(8):
                        i = i0 + b
                        a_t[i, :] = (
                            c1_t[i, :] * (g_a[i, :] + a_t[i, :]) + c2_t[i, :]
                        )

                pltpu.sync_copy(a_t, outa_sh.at[rows])

                @pl.loop(0, ROWS_PER_TILE // ZR)
                def _(t):
                    pltpu.sync_copy(zer_t, s_sh.at[pl.ds(rbase + t * ZR, ZR)])

                plsc.subcore_barrier()

            @pl.loop(0, ROWS_PER_TILE)
            def _(i):
                a_t[i, :] = a_t[i, :] / dis_t[i, :]

            pltpu.sync_copy(a_t, out_hbm.at[rows])

    return k(z, src_pad, dst_pad)


def kernel(x, edge_index, W1, b1, W2, b2):
    z = _mlp(x, W1.T, b1, W2.T, b2)
    z = jnp.pad(z, ((0, PAD_ROWS - N_NODES), (0, 0)))

    e = edge_index.shape[1]
    nchunk = -(-e // (NS * CB))
    nchunk += nchunk % 2
    ep = nchunk * NS * CB
    npad = ep - e
    pad_idx = N_NODES + (jnp.arange(npad, dtype=jnp.int32) % (PAD_ROWS - N_NODES))
    src_pad = jnp.concatenate([edge_index[0], pad_idx]).reshape(NS, nchunk, CB)
    dst_pad = jnp.concatenate([edge_index[1], pad_idx]).reshape(NS, nchunk, CB)

    return _appnp_sc(z, src_pad, dst_pad, nchunk)[:N_NODES]
